# baseline (speedup 1.0000x reference)
import jax
import jax.numpy as jnp
from jax import lax
from jax.experimental import pallas as pl
from jax.experimental.pallas import tpu as pltpu

_R = 20000
_P = 20480
_ROWS = 160
_QROWS = 40
_LANES = 128
_NGT = 50
_NIMG = 2
_PER_IMG = 256
_FG = 64
_BG = 192
_NCOL = 324
_NCOLP = 384
_BIG = 2 ** 30


def _kernel_body(rt_ref, rois8_ref, gt_ref, gtT_ref,
                 oroi_ref, olab_ref, obt_ref, obw_ref,
                 sf0_ref, sb0_ref, sf1_ref, sb1_ref, cmp0_ref, cmp1_ref,
                 bvf0_ref, bcf0_ref, bvb0_ref, bcb0_ref,
                 bvf1_ref, bcf1_ref, bvb1_ref, bcb1_ref):
    s_refs = ((sf0_ref, sb0_ref), (sf1_ref, sb1_ref))
    cmp_refs = (cmp0_ref, cmp1_ref)
    sum_refs = (((bvf0_ref, bcf0_ref), (bvb0_ref, bcb0_ref)),
                ((bvf1_ref, bcf1_ref), (bvb1_ref, bcb1_ref)))

    def pick(va, ca, vb, cb):
        take = (va > vb) | ((va == vb) & (ca < cb))
        return jnp.where(take, va, vb), jnp.where(take, ca, cb)

    def blk_tree(v, c):
        v, c = pick(v[0:4], c[0:4], v[4:8], c[4:8])
        v, c = pick(v[0:2], c[0:2], v[2:4], c[2:4])
        v, c = pick(v[0:1], c[0:1], v[1:2], c[1:2])
        return v, c

    idxq = (lax.broadcasted_iota(jnp.int32, (_QROWS, _LANES), 0) * _LANES
            + lax.broadcasted_iota(jnp.int32, (_QROWS, _LANES), 1))

    for b in range(_NIMG):
        for q in range(_ROWS // _QROWS):
            r0 = q * _QROWS
            bidxq = rt_ref[0, pl.ds(r0, _QROWS), :]
            x1q = rt_ref[1, pl.ds(r0, _QROWS), :]
            y1q = rt_ref[2, pl.ds(r0, _QROWS), :]
            x2q = rt_ref[3, pl.ds(r0, _QROWS), :]
            y2q = rt_ref[4, pl.ds(r0, _QROWS), :]
            area_a = (x2q - x1q + 1.0) * (y2q - y1q + 1.0)

            def gt_step(g, carry):
                m, am = carry
                gv = gt_ref[pl.ds(b * _NGT + g, 1), :]
                bx1 = gv[:, 0:1]
                by1 = gv[:, 1:2]
                bx2 = gv[:, 2:3]
                by2 = gv[:, 3:4]
                iw = jnp.maximum(
                    jnp.minimum(x2q, bx2) - jnp.maximum(x1q, bx1) + 1.0, 0.0)
                ih = jnp.maximum(
                    jnp.minimum(y2q, by2) - jnp.maximum(y1q, by1) + 1.0, 0.0)
                inter = iw * ih
                area_b = (bx2 - bx1 + 1.0) * (by2 - by1 + 1.0)
                union = area_a + area_b - inter
                ov = inter / union
                better = ov > m
                return jnp.where(better, ov, m), jnp.where(better, g, am)

            m, am = lax.fori_loop(
                0, _NGT, gt_step,
                (jnp.full((_QROWS, _LANES), -jnp.inf, jnp.float32),
                 jnp.zeros((_QROWS, _LANES), jnp.int32)))

            in_img = bidxq == jnp.float32(b)
            fgs = jnp.where(in_img & (m >= 0.5), m, -1.0)
            bgs = jnp.where(in_img & (m < 0.5), m, -1.0)
            cmpq = (idxq + r0 * _LANES) * 64 + am

            sf, sb = s_refs[b]
            sf[pl.ds(r0, _QROWS), :] = fgs
            sb[pl.ds(r0, _QROWS), :] = bgs
            cmp_refs[b][pl.ds(r0, _QROWS), :] = cmpq

            for kk in range(_QROWS // 8):
                krow = q * (_QROWS // 8) + kk
                for fgbg, grid in ((0, fgs), (1, bgs)):
                    v, c = blk_tree(grid[8 * kk:8 * kk + 8],
                                    cmpq[8 * kk:8 * kk + 8])
                    bv, bc = sum_refs[b][fgbg]
                    bv[pl.ds(krow, 1), :] = v
                    bc[pl.ds(krow, 1), :] = c

    olab_ref[...] = jnp.zeros((_NIMG * _PER_IMG, 8), jnp.float32)
    obt_ref[...] = jnp.zeros((_NIMG * _PER_IMG, _NCOLP), jnp.float32)
    obw_ref[...] = jnp.zeros((_NIMG * _PER_IMG, _NCOLP), jnp.float32)

    sub8 = lax.broadcasted_iota(jnp.int32, (8, _LANES), 0)
    lane8 = lax.broadcasted_iota(jnp.int32, (8, _LANES), 1)
    blkpos = sub8 * _LANES + lane8
    lane8r = lax.broadcasted_iota(jnp.int32, (1, 8), 1)

    def extract(b, fgbg):
        s_ref = s_refs[b][fgbg]
        comp_ref = cmp_refs[b]
        bv_ref, bc_ref = sum_refs[b][fgbg]
        bv = bv_ref[...]
        val = jnp.max(bv)
        selc = jnp.min(jnp.where(bv == val, bc_ref[...], _BIG))
        lin = selc >> 6
        ga = selc & 63
        blkbase = (lin >> 10) << 3
        sblk = s_ref[pl.ds(blkbase, 8), :]
        sblk = jnp.where(blkpos == (lin & 1023), -2.0, sblk)
        s_ref[pl.ds(blkbase, 8), :] = sblk
        cblk = comp_ref[pl.ds(blkbase, 8), :]
        v, c = blk_tree(sblk, cblk)
        bv_ref[pl.ds(blkbase >> 3, 1), :] = v
        bc_ref[pl.ds(blkbase >> 3, 1), :] = c
        return val, lin, ga

    def fg_store(b, i, val, sel, ga):
        rv = rois8_ref[pl.ds(sel, 1), :]
        rvx = jnp.where(lane8r == 5, val,
                        jnp.where(lane8r == 6, ga.astype(jnp.float32), rv))
        oroi_ref[pl.ds(b * _PER_IMG + i, 1), :] = rvx

    def bg_store(b, i, sel):
        rv = rois8_ref[pl.ds(sel, 1), :]
        oroi_ref[pl.ds(b * _PER_IMG + _FG + i, 1), :] = rv

    def body1(i, carry):
        v0, s0, a0 = extract(0, 0)
        v1, s1, a1 = extract(1, 0)
        _, sb0, _ = extract(0, 1)
        _, sb1, _ = extract(1, 1)
        fg_store(0, i, v0, s0, a0)
        fg_store(1, i, v1, s1, a1)
        bg_store(0, i, sb0)
        bg_store(1, i, sb1)
        return carry

    lax.fori_loop(0, _FG, body1, 0)

    def body2(i, carry):
        _, sb0, _ = extract(0, 1)
        _, sb1, _ = extract(1, 1)
        bg_store(0, i, sb0)
        bg_store(1, i, sb1)
        return carry

    lax.fori_loop(_FG, _BG, body2, 0)

    F = jnp.concatenate(
        [oroi_ref[0:_FG, :], oroi_ref[_PER_IMG:_PER_IMG + _FG, :]], 0)
    val_c = F[:, 5:6]
    ga_c = F[:, 6:7].astype(jnp.int32)
    valid = val_c > 0.0
    fgf = valid.astype(jnp.float32)
    riota = lax.broadcasted_iota(jnp.int32, (2 * _FG, 1), 0)
    gidx = ga_c + jnp.where(riota < _FG, 0, _NGT)
    onehot = (gidx == lax.broadcasted_iota(
        jnp.int32, (2 * _FG, _NIMG * _NGT), 1)).astype(jnp.float32)

    def gcol(c):
        return jnp.sum(onehot * gtT_ref[c:c + 1, :], axis=1, keepdims=True)

    gx1, gy1, gx2, gy2, glab = gcol(0), gcol(1), gcol(2), gcol(3), gcol(4)
    label = jnp.where(valid, glab, 0.0)
    ex_w = F[:, 3:4] - F[:, 1:2] + 1.0
    ex_h = F[:, 4:5] - F[:, 2:3] + 1.0
    ex_cx = F[:, 1:2] + 0.5 * ex_w
    ex_cy = F[:, 2:3] + 0.5 * ex_h
    gt_w = gx2 - gx1 + 1.0
    gt_h = gy2 - gy1 + 1.0
    gt_cx = gx1 + 0.5 * gt_w
    gt_cy = gy1 + 0.5 * gt_h
    dx = (gt_cx - ex_cx) / ex_w
    dy = (gt_cy - ex_cy) / ex_h
    dw = jnp.log(gt_w / ex_w)
    dh = jnp.log(gt_h / ex_h)
    cif = lax.broadcasted_iota(jnp.int32, (2 * _FG, _NCOLP), 1)
    cls = label.astype(jnp.int32)
    maskc = (cif >> 2) == cls
    j = cif & 3
    tsel = jnp.where(j == 0, dx,
                     jnp.where(j == 1, dy, jnp.where(j == 2, dw, dh)))
    btF = jnp.where(maskc, tsel * fgf, 0.0)
    bwF = jnp.where(maskc, fgf * jnp.ones_like(tsel), 0.0)
    obt_ref[0:_FG, :] = btF[0:_FG]
    obt_ref[_PER_IMG:_PER_IMG + _FG, :] = btF[_FG:]
    obw_ref[0:_FG, :] = bwF[0:_FG]
    obw_ref[_PER_IMG:_PER_IMG + _FG, :] = bwF[_FG:]
    olab_ref[0:_FG, :] = jnp.broadcast_to(label[0:_FG], (_FG, 8))
    olab_ref[_PER_IMG:_PER_IMG + _FG, :] = jnp.broadcast_to(label[_FG:], (_FG, 8))


def _build_call(interpret=False):
    return pl.pallas_call(
        _kernel_body,
        out_shape=[
            jax.ShapeDtypeStruct((_NIMG * _PER_IMG, 8), jnp.float32),
            jax.ShapeDtypeStruct((_NIMG * _PER_IMG, 8), jnp.float32),
            jax.ShapeDtypeStruct((_NIMG * _PER_IMG, _NCOLP), jnp.float32),
            jax.ShapeDtypeStruct((_NIMG * _PER_IMG, _NCOLP), jnp.float32),
        ],
        scratch_shapes=(
            [pltpu.VMEM((_ROWS, _LANES), jnp.float32) for _ in range(4)]
            + [pltpu.VMEM((_ROWS, _LANES), jnp.int32) for _ in range(2)]
            + [pltpu.VMEM((_ROWS // 8, _LANES),
                          jnp.float32 if i % 2 == 0 else jnp.int32)
               for i in range(8)]
        ),
        interpret=interpret,
    )


def kernel(rois, gt_boxes):
    rt = jnp.pad(rois.T, ((0, 0), (0, _P - _R)), constant_values=-1.0)
    rt = rt.reshape(5, _ROWS, _LANES)
    rois8 = jnp.pad(rois, ((0, 0), (0, 3)))
    gt2 = jnp.pad(gt_boxes.reshape(_NIMG * _NGT, 5), ((0, 0), (0, 3)))
    gtT = gt_boxes.reshape(_NIMG * _NGT, 5).T
    oroi, olab, obt, obw = _build_call()(rt, rois8, gt2, gtT)
    return oroi[:, :5], olab[:, 0], obt[:, :_NCOL], obw[:, :_NCOL]
